# Initial kernel scaffold; baseline (speedup 1.0000x reference)
import jax
import jax.numpy as jnp
from jax.experimental import pallas as pl


def kernel(token_ids, embedding):
    raise NotImplementedError("write your pallas kernel here")


# speedup vs baseline: 2.9690x; 2.9690x over previous
import functools

import jax
import jax.numpy as jnp
from jax import lax
from jax.experimental import pallas as pl
from jax.experimental.pallas import tpu as pltpu
from jax.experimental.pallas import tpu_sc as plsc

_NUM_CORES = 2
_NUM_SUBCORES = 16
_NW = _NUM_CORES * _NUM_SUBCORES
_CHUNK = 128


def _gather_body(nchunk, per_w, idx_hbm, table_hbm, out_hbm, idx_v, rows_v, sem):
    wid = lax.axis_index("s") * _NUM_CORES + lax.axis_index("c")
    pltpu.sync_copy(idx_hbm.at[wid], idx_v)
    base = wid * per_w

    def step(j, carry):
        pltpu.async_copy(table_hbm.at[idx_v.at[j]], rows_v, sem).wait()
        off = pl.multiple_of(base + j * _CHUNK, _CHUNK)
        pltpu.sync_copy(rows_v, out_hbm.at[pl.ds(off, _CHUNK)])
        return carry

    lax.fori_loop(0, nchunk, step, 0)


@functools.partial(jax.jit, static_argnums=(2, 3))
def _gather(idx, table, n_flat, d):
    per_w = n_flat // _NW
    nchunk = per_w // _CHUNK
    mesh = plsc.VectorSubcoreMesh(core_axis_name="c", subcore_axis_name="s")
    f = pl.kernel(
        functools.partial(_gather_body, nchunk, per_w),
        out_type=jax.ShapeDtypeStruct((n_flat, d), jnp.float32),
        mesh=mesh,
        scratch_types=[
            pltpu.VMEM((nchunk, _CHUNK), jnp.int32),
            pltpu.VMEM((_CHUNK, d), jnp.float32),
            pltpu.SemaphoreType.DMA,
        ],
    )
    return f(idx, table)


def kernel(token_ids, embedding):
    n_flat = token_ids.size
    d = embedding.shape[1]
    per_w = n_flat // _NW
    nchunk = per_w // _CHUNK
    idx = token_ids.reshape(_NW, nchunk, _CHUNK).astype(jnp.int32)
    out = _gather(idx, embedding, n_flat, d)
    return out.reshape(*token_ids.shape, d)


# speedup vs baseline: 3.1357x; 1.0561x over previous
import functools

import jax
import jax.numpy as jnp
from jax import lax
from jax.experimental import pallas as pl
from jax.experimental.pallas import tpu as pltpu
from jax.experimental.pallas import tpu_sc as plsc

_NUM_CORES = 2
_NUM_SUBCORES = 16
_NW = _NUM_CORES * _NUM_SUBCORES
_CHUNK = 128


def _gather_body(nchunk, per_w, idx_hbm, table_hbm, out_hbm, idx_v,
                 rows0, rows1, gsem0, gsem1, osem0, osem1):
    wid = lax.axis_index("s") * _NUM_CORES + lax.axis_index("c")
    pltpu.sync_copy(idx_hbm.at[wid], idx_v)
    base = wid * per_w

    rows = (rows0, rows1)
    gsem = (gsem0, gsem1)
    osem = (osem0, osem1)

    def out_ref(j):
        off = pl.multiple_of(base + j * _CHUNK, _CHUNK)
        return out_hbm.at[pl.ds(off, _CHUNK)]

    def start_gather(j, b):
        pltpu.async_copy(table_hbm.at[idx_v.at[j]], rows[b], gsem[b])

    def wait_gather(j, b):
        pltpu.make_async_copy(table_hbm.at[idx_v.at[j]], rows[b], gsem[b]).wait()

    def start_out(j, b):
        pltpu.async_copy(rows[b], out_ref(j), osem[b])

    def wait_out(j, b):
        pltpu.make_async_copy(rows[b], out_ref(j), osem[b]).wait()

    npair = nchunk // 2
    start_gather(0, 0)

    def step(i, carry):
        j0 = i * 2
        wait_gather(j0, 0)

        @pl.when(i > 0)
        def _():
            wait_out(j0 - 1, 1)

        start_gather(j0 + 1, 1)
        start_out(j0, 0)

        wait_gather(j0 + 1, 1)
        wait_out(j0, 0)

        @pl.when(i < npair - 1)
        def _():
            start_gather(j0 + 2, 0)

        start_out(j0 + 1, 1)
        return carry

    lax.fori_loop(0, npair, step, 0)
    wait_out(nchunk - 1, 1)


@functools.partial(jax.jit, static_argnums=(2, 3))
def _gather(idx, table, n_flat, d):
    per_w = n_flat // _NW
    nchunk = per_w // _CHUNK
    mesh = plsc.VectorSubcoreMesh(core_axis_name="c", subcore_axis_name="s")
    f = pl.kernel(
        functools.partial(_gather_body, nchunk, per_w),
        out_type=jax.ShapeDtypeStruct((n_flat, d), jnp.float32),
        mesh=mesh,
        scratch_types=[
            pltpu.VMEM((nchunk, _CHUNK), jnp.int32),
            pltpu.VMEM((_CHUNK, d), jnp.float32),
            pltpu.VMEM((_CHUNK, d), jnp.float32),
            pltpu.SemaphoreType.DMA,
            pltpu.SemaphoreType.DMA,
            pltpu.SemaphoreType.DMA,
            pltpu.SemaphoreType.DMA,
        ],
    )
    return f(idx, table)


def kernel(token_ids, embedding):
    n_flat = token_ids.size
    d = embedding.shape[1]
    per_w = n_flat // _NW
    nchunk = per_w // _CHUNK
    idx = token_ids.reshape(_NW, nchunk, _CHUNK).astype(jnp.int32)
    out = _gather(idx, embedding, n_flat, d)
    return out.reshape(*token_ids.shape, d)


# speedup vs baseline: 3.3323x; 1.0627x over previous
import functools

import jax
import jax.numpy as jnp
from jax import lax
from jax.experimental import pallas as pl
from jax.experimental.pallas import tpu as pltpu
from jax.experimental.pallas import tpu_sc as plsc

_NUM_CORES = 2
_NUM_SUBCORES = 16
_NW = _NUM_CORES * _NUM_SUBCORES
_CHUNK = 128


_NBUF = 5


def _gather_body(nchunk, per_w, idx_hbm, table_hbm, out_hbm, idx_v,
                 *scratch):
    rows = scratch[:_NBUF]
    gsem = scratch[_NBUF:2 * _NBUF]
    osem = scratch[2 * _NBUF:3 * _NBUF]
    wid = lax.axis_index("s") * _NUM_CORES + lax.axis_index("c")
    pltpu.sync_copy(idx_hbm.at[wid], idx_v)
    base = wid * per_w

    def out_ref(j):
        off = pl.multiple_of(base + j * _CHUNK, _CHUNK)
        return out_hbm.at[pl.ds(off, _CHUNK)]

    def start_gather(j, b):
        pltpu.async_copy(table_hbm.at[idx_v.at[j]], rows[b], gsem[b])

    def wait_gather(j, b):
        pltpu.make_async_copy(table_hbm.at[idx_v.at[j]], rows[b], gsem[b]).wait()

    def start_out(j, b):
        pltpu.async_copy(rows[b], out_ref(j), osem[b])

    def wait_out(j, b):
        pltpu.make_async_copy(rows[b], out_ref(j), osem[b]).wait()

    for b in range(_NBUF - 1):
        start_gather(b, b)

    ngroup = nchunk // _NBUF

    def step(i, carry):
        for b in range(_NBUF):
            j = i * _NBUF + b
            wait_gather(j, b)
            start_out(j, b)
            bp = (b - 1) % _NBUF
            jn = j + _NBUF - 1

            @pl.when(j > 0)
            def _():
                wait_out(j - 1, bp)

            @pl.when(jn < nchunk)
            def _():
                start_gather(jn, bp)
        return carry

    lax.fori_loop(0, ngroup, step, 0)
    wait_out(nchunk - 1, (nchunk - 1) % _NBUF)


@functools.partial(jax.jit, static_argnums=(2, 3))
def _gather(idx, table, n_flat, d):
    per_w = n_flat // _NW
    nchunk = per_w // _CHUNK
    mesh = plsc.VectorSubcoreMesh(core_axis_name="c", subcore_axis_name="s")
    f = pl.kernel(
        functools.partial(_gather_body, nchunk, per_w),
        out_type=jax.ShapeDtypeStruct((n_flat, d), jnp.float32),
        mesh=mesh,
        scratch_types=(
            [pltpu.VMEM((nchunk, _CHUNK), jnp.int32)]
            + [pltpu.VMEM((_CHUNK, d), jnp.float32)] * _NBUF
            + [pltpu.SemaphoreType.DMA] * (2 * _NBUF)
        ),
    )
    return f(idx, table)


def kernel(token_ids, embedding):
    n_flat = token_ids.size
    d = embedding.shape[1]
    per_w = n_flat // _NW
    nchunk = per_w // _CHUNK
    idx = token_ids.reshape(_NW, nchunk, _CHUNK).astype(jnp.int32)
    out = _gather(idx, embedding, n_flat, d)
    return out.reshape(*token_ids.shape, d)


# speedup vs baseline: 5.9810x; 1.7949x over previous
import functools

import jax
import jax.numpy as jnp
from jax import lax
from jax.experimental import pallas as pl
from jax.experimental.pallas import tpu as pltpu
from jax.experimental.pallas import tpu_sc as plsc

_NUM_CORES = 2
_NUM_SUBCORES = 16
_NW = _NUM_CORES * _NUM_SUBCORES
_IDX_PER_STREAM = 100
_SLAB_ROWS = 8
_NBUF = 2


def _gather_body(nslab, rows_per_w, n_tok, idx_hbm, table_hbm, out_hbm, idx_v,
                 *scratch):
    slabs = scratch[:_NBUF]
    gsem = scratch[_NBUF:2 * _NBUF]
    osem = scratch[2 * _NBUF:3 * _NBUF]
    wid = lax.axis_index("s") * _NUM_CORES + lax.axis_index("c")
    pltpu.sync_copy(idx_hbm.at[wid], idx_v)
    base_row = wid * rows_per_w

    slab_flat = _SLAB_ROWS * n_tok
    ng = slab_flat // _IDX_PER_STREAM

    def g_descr(i, g, b):
        src = table_hbm.at[idx_v.at[i * ng + g]]
        dst = slabs[b].at[pl.ds(g * _IDX_PER_STREAM, _IDX_PER_STREAM)]
        return src, dst

    def fire_gathers(i, b):
        for g in range(ng):
            src, dst = g_descr(i, g, b)
            pltpu.async_copy(src, dst, gsem[b])

    def wait_gathers(i, b):
        for g in range(ng):
            src, dst = g_descr(i, g, b)
            pltpu.make_async_copy(src, dst, gsem[b]).wait()

    def o_descr(i, r, b):
        src = slabs[b].at[pl.ds(r * n_tok, n_tok)]
        dst = out_hbm.at[base_row + i * _SLAB_ROWS + r]
        return src, dst

    def fire_outs(i, b):
        for r in range(_SLAB_ROWS):
            src, dst = o_descr(i, r, b)
            pltpu.async_copy(src, dst, osem[b])

    def wait_outs(i, b):
        for r in range(_SLAB_ROWS):
            src, dst = o_descr(i, r, b)
            pltpu.make_async_copy(src, dst, osem[b]).wait()

    fire_gathers(0, 0)

    def step(i, carry):
        for b in range(_NBUF):
            j = i * _NBUF + b
            jn = j + 1
            bn = (b + 1) % _NBUF

            @pl.when(j > 0)
            def _():
                wait_outs(j - 1, bn)

            @pl.when(jn < nslab)
            def _():
                fire_gathers(jn, bn)

            wait_gathers(j, b)
            fire_outs(j, b)
        return carry

    lax.fori_loop(0, nslab // _NBUF, step, 0)
    wait_outs(nslab - 1, (nslab - 1) % _NBUF)


@functools.partial(jax.jit, static_argnums=(2, 3, 4))
def _gather(idx, table, n_batch, n_tok, d):
    rows_per_w = n_batch // _NW
    nslab = rows_per_w // _SLAB_ROWS
    nstream = rows_per_w * n_tok // _IDX_PER_STREAM
    mesh = plsc.VectorSubcoreMesh(core_axis_name="c", subcore_axis_name="s")
    f = pl.kernel(
        functools.partial(_gather_body, nslab, rows_per_w, n_tok),
        out_type=jax.ShapeDtypeStruct((n_batch, n_tok, d), jnp.float32),
        mesh=mesh,
        scratch_types=(
            [pltpu.VMEM((nstream, _IDX_PER_STREAM), jnp.int32)]
            + [pltpu.VMEM((_SLAB_ROWS * n_tok, d), jnp.float32)] * _NBUF
            + [pltpu.SemaphoreType.DMA] * (2 * _NBUF)
        ),
    )
    return f(idx, table)


def kernel(token_ids, embedding):
    n_batch, n_tok = token_ids.shape
    d = embedding.shape[1]
    rows_per_w = n_batch // _NW
    nstream = rows_per_w * n_tok // _IDX_PER_STREAM
    idx = token_ids.reshape(_NW, nstream, _IDX_PER_STREAM).astype(jnp.int32)
    return _gather(idx, embedding, n_batch, n_tok, d)


# speedup vs baseline: 6.0129x; 1.0053x over previous
import functools

import jax
import jax.numpy as jnp
from jax import lax
from jax.experimental import pallas as pl
from jax.experimental.pallas import tpu as pltpu
from jax.experimental.pallas import tpu_sc as plsc

_NUM_CORES = 2
_NUM_SUBCORES = 16
_NW = _NUM_CORES * _NUM_SUBCORES
_IPS = 100
_NBUF = 8


def _gather_body(nstream, n_tok, idx_hbm, table_hbm, out_hbm, idx_v, *scratch):
    slots = scratch[:_NBUF]
    gsem = scratch[_NBUF:2 * _NBUF]
    osem = scratch[2 * _NBUF:3 * _NBUF]
    wid = lax.axis_index("s") * _NUM_CORES + lax.axis_index("c")
    pltpu.sync_copy(idx_hbm.at[wid], idx_v)
    rows_per_stream = _IPS // n_tok
    base_row = wid * nstream * rows_per_stream

    def g_descr(j, b):
        return table_hbm.at[idx_v.at[j]], slots[b]

    def o_descr(j, r, b):
        src = slots[b].at[pl.ds(r * n_tok, n_tok)]
        dst = out_hbm.at[base_row + j * rows_per_stream + r]
        return src, dst

    def fire_gather(j, b):
        src, dst = g_descr(j, b)
        pltpu.async_copy(src, dst, gsem[b])

    def wait_gather(j, b):
        src, dst = g_descr(j, b)
        pltpu.make_async_copy(src, dst, gsem[b]).wait()

    def fire_outs(j, b):
        for r in range(rows_per_stream):
            src, dst = o_descr(j, r, b)
            pltpu.async_copy(src, dst, osem[b])

    def wait_outs(j, b):
        for r in range(rows_per_stream):
            src, dst = o_descr(j, r, b)
            pltpu.make_async_copy(src, dst, osem[b]).wait()

    for b in range(_NBUF - 1):
        fire_gather(b, b)

    def step(i, carry):
        for b in range(_NBUF):
            j = i * _NBUF + b
            bp = (b - 1) % _NBUF
            jn = j + _NBUF - 1

            @pl.when(j > 0)
            def _():
                wait_outs(j - 1, bp)

            @pl.when(jn < nstream)
            def _():
                fire_gather(jn, bp)

            wait_gather(j, b)
            fire_outs(j, b)
        return carry

    lax.fori_loop(0, nstream // _NBUF, step, 0)
    wait_outs(nstream - 1, (nstream - 1) % _NBUF)


@functools.partial(jax.jit, static_argnums=(2, 3, 4))
def _gather(idx, table, n_batch, n_tok, d):
    nstream = n_batch * n_tok // _NW // _IPS
    mesh = plsc.VectorSubcoreMesh(core_axis_name="c", subcore_axis_name="s")
    f = pl.kernel(
        functools.partial(_gather_body, nstream, n_tok),
        out_type=jax.ShapeDtypeStruct((n_batch, n_tok, d), jnp.float32),
        mesh=mesh,
        scratch_types=(
            [pltpu.VMEM((nstream, _IPS), jnp.int32)]
            + [pltpu.VMEM((_IPS, d), jnp.float32)] * _NBUF
            + [pltpu.SemaphoreType.DMA] * (2 * _NBUF)
        ),
    )
    return f(idx, table)


def kernel(token_ids, embedding):
    n_batch, n_tok = token_ids.shape
    d = embedding.shape[1]
    nstream = n_batch * n_tok // _NW // _IPS
    idx = token_ids.reshape(_NW, nstream, _IPS).astype(jnp.int32)
    return _gather(idx, embedding, n_batch, n_tok, d)
